# baseline (speedup 1.0000x reference)
import jax
import jax.numpy as jnp
from jax import lax
from jax.experimental import pallas as pl
from jax.experimental.pallas import tpu as pltpu
from jax.experimental.pallas import tpu_sc as plsc

_NUM_CLASSES = 1000
_FEAT = 2048
_N = 8192
_SIGMA = 0.2

_NW = 32
_COLS = 64
_CG = _COLS // 16
_R = 128
_NCH = _N // _R
_NP = _NCH // 2


def _zero_acc(acc):
    z = jnp.zeros((16,), jnp.float32)

    def body(r, _):
        for g in range(_CG):
            acc[pl.ds(r * _COLS + g * 16, 16)] = z
        return 0
    lax.fori_loop(0, _NUM_CLASSES, body, 0)


def _feat_slice(feats, j, slice0):
    return feats.at[pl.ds(j * _R, _R), pl.ds(slice0, 128)]


def _accumulate_chunk(acc, fb, lab, base, coff):
    def rowblk(rb, _):
        lv = lab[pl.ds(base + rb * 16, 16)]
        for r in range(16):
            cbase = lv[r] * _COLS
            for g in range(_CG):
                plsc.addupdate(acc.at[pl.ds(cbase + g * 16, 16)],
                               fb[rb * 16 + r, pl.ds(coff + g * 16, 16)])
        return 0
    lax.fori_loop(0, _R // 16, rowblk, 0)


def _modality(feats, labels, out_hbm, lab, acc, fb0, fb1,
              sem0, sem1, sem2, w, slice0, coff):
    pltpu.async_copy(labels, lab, sem2)
    _zero_acc(acc)
    pltpu.make_async_copy(labels, lab, sem2).wait()
    pltpu.async_copy(_feat_slice(feats, 0, slice0), fb0, sem0)

    def pair(p, _):
        pltpu.async_copy(_feat_slice(feats, 2 * p + 1, slice0), fb1, sem1)
        pltpu.make_async_copy(_feat_slice(feats, 2 * p, slice0), fb0,
                              sem0).wait()
        _accumulate_chunk(acc, fb0, lab, 2 * p * _R, coff)

        @pl.when(p + 1 < _NP)
        def _():
            pltpu.async_copy(_feat_slice(feats, 2 * p + 2, slice0), fb0, sem0)
        pltpu.make_async_copy(_feat_slice(feats, 2 * p + 1, slice0), fb1,
                              sem1).wait()
        _accumulate_chunk(acc, fb1, lab, (2 * p + 1) * _R, coff)
        return 0
    lax.fori_loop(0, _NP, pair, 0)
    pltpu.sync_copy(acc, out_hbm.at[pl.ds(w * _NUM_CLASSES * _COLS,
                                          _NUM_CLASSES * _COLS)])


def _segsum_body(rgb_f, ir_f, rgb_l, ir_l, acc_v_hbm, acc_i_hbm,
                 lab, fb0, fb1, acc, sem0, sem1, sem2):
    c = lax.axis_index("c")
    s = lax.axis_index("s")
    w = s * 2 + c
    slice0 = (w // 2) * 128
    coff = (w % 2) * 64
    _modality(rgb_f, rgb_l, acc_v_hbm, lab, acc, fb0, fb1,
              sem0, sem1, sem2, w, slice0, coff)
    _modality(ir_f, ir_l, acc_i_hbm, lab, acc, fb0, fb1,
              sem0, sem1, sem2, w, slice0, coff)


def _segsum(rgb_feats, ir_feats, rgb_labels, ir_labels):
    mesh = plsc.VectorSubcoreMesh(core_axis_name="c", subcore_axis_name="s")
    run = pl.kernel(
        _segsum_body,
        out_type=(
            jax.ShapeDtypeStruct((_NW * _NUM_CLASSES * _COLS,), jnp.float32),
            jax.ShapeDtypeStruct((_NW * _NUM_CLASSES * _COLS,), jnp.float32),
        ),
        mesh=mesh,
        scratch_types=[
            pltpu.VMEM((_N,), jnp.int32),
            pltpu.VMEM((_R, 128), jnp.float32),
            pltpu.VMEM((_R, 128), jnp.float32),
            pltpu.VMEM((_NUM_CLASSES * _COLS,), jnp.float32),
            pltpu.SemaphoreType.DMA,
            pltpu.SemaphoreType.DMA,
            pltpu.SemaphoreType.DMA,
        ],
    )
    return run(rgb_feats, ir_feats, rgb_labels, ir_labels)


_LCH = 512
_NLCH = _N // _LCH


def _bincount_body(lab_ref, out_ref):
    cls = lax.broadcasted_iota(jnp.int32, (1024, 1), 0)
    for m in range(2):
        def body(i, acc, m=m):
            row = lab_ref[m, i]
            eq = (cls == row[None, :]).astype(jnp.float32)
            return acc + jnp.sum(eq, axis=1, keepdims=True)
        acc = lax.fori_loop(0, _NLCH, body,
                            jnp.zeros((1024, 1), jnp.float32))
        out_ref[m] = acc


def _bincount(rgb_labels, ir_labels):
    labs = jnp.stack([rgb_labels, ir_labels]).reshape(2, _NLCH, _LCH)
    return pl.pallas_call(
        _bincount_body,
        out_shape=jax.ShapeDtypeStruct((2, 1024, 1), jnp.float32),
    )(labs)


_RB = 200
_NRB = _NUM_CLASSES // _RB
_NWP = _NW // 2


def _ema_body(acc_v_ref, acc_i_ref, mem_v_ref, mem_i_ref, cnt_ref, out_ref):
    for m, (acc_ref, mem_ref) in enumerate(
            ((acc_v_ref, mem_v_ref), (acc_i_ref, mem_i_ref))):
        acc = jnp.concatenate([acc_ref[0], acc_ref[1]], axis=1)
        mem = mem_ref[...]
        cnt = cnt_ref[m]
        factor = _SIGMA / jnp.maximum(cnt, 1.0)
        upd = mem * (1.0 - _SIGMA) + acc * factor
        out_ref[m] = jnp.where(cnt > 0.0, upd, mem)


def _ema(acc_v, acc_i, vis_memory, ir_memory, cnts):
    acc_spec = pl.BlockSpec((2, _RB, _COLS), lambda b, wp: (wp, b, 0))
    mem_spec = pl.BlockSpec((_RB, 128), lambda b, wp: (b, wp))
    return pl.pallas_call(
        _ema_body,
        grid=(_NRB, _NWP),
        in_specs=[
            acc_spec,
            acc_spec,
            mem_spec,
            mem_spec,
            pl.BlockSpec((2, _RB, 1), lambda b, wp: (0, b, 0)),
        ],
        out_specs=pl.BlockSpec((2, _RB, 128), lambda b, wp: (0, b, wp)),
        out_shape=jax.ShapeDtypeStruct((2, _NUM_CLASSES, _FEAT), jnp.float32),
    )(acc_v, acc_i, vis_memory, ir_memory, cnts)


@jax.jit
def _cma(rgb_feats, ir_feats, vis_memory, ir_memory, rgb_labels, ir_labels):
    acc_v, acc_i = _segsum(rgb_feats, ir_feats, rgb_labels, ir_labels)
    acc_v = acc_v.reshape(_NW, _NUM_CLASSES, _COLS)
    acc_i = acc_i.reshape(_NW, _NUM_CLASSES, _COLS)
    cnts = _bincount(rgb_labels, ir_labels)
    return _ema(acc_v, acc_i, vis_memory, ir_memory, cnts)


def kernel(rgb_feats, ir_feats, vis_memory, ir_memory, rgb_labels, ir_labels):
    return _cma(rgb_feats, ir_feats, vis_memory, ir_memory,
                rgb_labels.astype(jnp.int32), ir_labels.astype(jnp.int32))


# speedup vs baseline: 1.3722x; 1.3722x over previous
import jax
import jax.numpy as jnp
from jax import lax
from jax.experimental import pallas as pl
from jax.experimental.pallas import tpu as pltpu
from jax.experimental.pallas import tpu_sc as plsc

_NUM_CLASSES = 1000
_FEAT = 2048
_N = 8192
_SIGMA = 0.2

_NW = 32
_COLS = 64
_CG = _COLS // 16
_STR = 80
_R = 128
_NCH = _N // _R
_NP = _NCH // 2


def _zero_acc(acc):
    z = jnp.zeros((16,), jnp.float32)

    def body(r, _):
        for g in range(_CG + 1):
            acc[pl.ds(r * _STR + g * 16, 16)] = z
        return 0
    lax.fori_loop(0, _NUM_CLASSES, body, 0)


def _feat_slice(feats, j, slice0):
    return feats.at[pl.ds(j * _R, _R), pl.ds(slice0, 128)]


def _accumulate_chunk(acc, fb, lab, base, coff):
    one = jnp.ones((16,), jnp.float32)

    def rowblk(rb, _):
        lv = lab[pl.ds(base + rb * 16, 16)]
        cbase = [lv[r] * _STR for r in range(16)]
        def load4(r4):
            return [[fb[rb * 16 + 4 * r4 + i, pl.ds(coff + g * 16, 16)]
                     for g in range(_CG)] for i in range(4)]

        def store4(r4, vals):
            for i in range(4):
                r = 4 * r4 + i
                for g in range(_CG):
                    plsc.addupdate(acc.at[pl.ds(cbase[r] + g * 16, 16)],
                                   vals[i][g])
                plsc.addupdate(acc.at[pl.ds(cbase[r] + _COLS, 16)], one)

        vals = load4(0)
        for r4 in range(1, 4):
            nxt = load4(r4)
            store4(r4 - 1, vals)
            vals = nxt
        store4(3, vals)
        return 0
    lax.fori_loop(0, _R // 16, rowblk, 0)


def _modality(feats, labels, out_hbm, lab, acc, fb0, fb1,
              sem0, sem1, sem2, w, slice0, coff):
    pltpu.async_copy(labels, lab, sem2)
    _zero_acc(acc)
    pltpu.make_async_copy(labels, lab, sem2).wait()
    pltpu.async_copy(_feat_slice(feats, 0, slice0), fb0, sem0)

    def pair(p, _):
        pltpu.async_copy(_feat_slice(feats, 2 * p + 1, slice0), fb1, sem1)
        pltpu.make_async_copy(_feat_slice(feats, 2 * p, slice0), fb0,
                              sem0).wait()
        _accumulate_chunk(acc, fb0, lab, 2 * p * _R, coff)

        @pl.when(p + 1 < _NP)
        def _():
            pltpu.async_copy(_feat_slice(feats, 2 * p + 2, slice0), fb0, sem0)
        pltpu.make_async_copy(_feat_slice(feats, 2 * p + 1, slice0), fb1,
                              sem1).wait()
        _accumulate_chunk(acc, fb1, lab, (2 * p + 1) * _R, coff)
        return 0
    lax.fori_loop(0, _NP, pair, 0)
    pltpu.sync_copy(acc, out_hbm.at[pl.ds(w * _NUM_CLASSES * _STR,
                                          _NUM_CLASSES * _STR)])


def _segsum_body(rgb_f, ir_f, rgb_l, ir_l, acc_v_hbm, acc_i_hbm,
                 lab, fb0, fb1, acc, sem0, sem1, sem2):
    c = lax.axis_index("c")
    s = lax.axis_index("s")
    w = s * 2 + c
    slice0 = (w // 2) * 128
    coff = (w % 2) * 64
    _modality(rgb_f, rgb_l, acc_v_hbm, lab, acc, fb0, fb1,
              sem0, sem1, sem2, w, slice0, coff)
    _modality(ir_f, ir_l, acc_i_hbm, lab, acc, fb0, fb1,
              sem0, sem1, sem2, w, slice0, coff)


def _segsum(rgb_feats, ir_feats, rgb_labels, ir_labels):
    mesh = plsc.VectorSubcoreMesh(core_axis_name="c", subcore_axis_name="s")
    run = pl.kernel(
        _segsum_body,
        out_type=(
            jax.ShapeDtypeStruct((_NW * _NUM_CLASSES * _STR,), jnp.float32),
            jax.ShapeDtypeStruct((_NW * _NUM_CLASSES * _STR,), jnp.float32),
        ),
        mesh=mesh,
        scratch_types=[
            pltpu.VMEM((_N,), jnp.int32),
            pltpu.VMEM((_R, 128), jnp.float32),
            pltpu.VMEM((_R, 128), jnp.float32),
            pltpu.VMEM((_NUM_CLASSES * _STR,), jnp.float32),
            pltpu.SemaphoreType.DMA,
            pltpu.SemaphoreType.DMA,
            pltpu.SemaphoreType.DMA,
        ],
    )
    return run(rgb_feats, ir_feats, rgb_labels, ir_labels)


_RB = 200
_NRB = _NUM_CLASSES // _RB
_NWP = _NW // 2


def _ema_body(acc_v_ref, acc_i_ref, mem_v_ref, mem_i_ref, out_ref):
    for m, (acc_ref, mem_ref) in enumerate(
            ((acc_v_ref, mem_v_ref), (acc_i_ref, mem_i_ref))):
        acc = jnp.concatenate([acc_ref[0, :, :_COLS],
                               acc_ref[1, :, :_COLS]], axis=1)
        mem = mem_ref[...]
        cnt = acc_ref[0, :, _COLS:_COLS + 1]
        factor = _SIGMA / jnp.maximum(cnt, 1.0)
        upd = mem * (1.0 - _SIGMA) + acc * factor
        out_ref[m] = jnp.where(cnt > 0.0, upd, mem)


def _ema(acc_v, acc_i, vis_memory, ir_memory):
    acc_spec = pl.BlockSpec((2, _RB, _STR), lambda b, wp: (wp, b, 0))
    mem_spec = pl.BlockSpec((_RB, 128), lambda b, wp: (b, wp))
    return pl.pallas_call(
        _ema_body,
        grid=(_NRB, _NWP),
        in_specs=[
            acc_spec,
            acc_spec,
            mem_spec,
            mem_spec,
        ],
        out_specs=pl.BlockSpec((2, _RB, 128), lambda b, wp: (0, b, wp)),
        out_shape=jax.ShapeDtypeStruct((2, _NUM_CLASSES, _FEAT), jnp.float32),
    )(acc_v, acc_i, vis_memory, ir_memory)


@jax.jit
def _cma(rgb_feats, ir_feats, vis_memory, ir_memory, rgb_labels, ir_labels):
    acc_v, acc_i = _segsum(rgb_feats, ir_feats, rgb_labels, ir_labels)
    acc_v = acc_v.reshape(_NW, _NUM_CLASSES, _STR)
    acc_i = acc_i.reshape(_NW, _NUM_CLASSES, _STR)
    return _ema(acc_v, acc_i, vis_memory, ir_memory)


def kernel(rgb_feats, ir_feats, vis_memory, ir_memory, rgb_labels, ir_labels):
    return _cma(rgb_feats, ir_feats, vis_memory, ir_memory,
                rgb_labels.astype(jnp.int32), ir_labels.astype(jnp.int32))


# speedup vs baseline: 1.5671x; 1.1421x over previous
import jax
import jax.numpy as jnp
from jax import lax
from jax.experimental import pallas as pl
from jax.experimental.pallas import tpu as pltpu
from jax.experimental.pallas import tpu_sc as plsc

_NUM_CLASSES = 1000
_FEAT = 2048
_N = 8192
_SIGMA = 0.2

_NW = 32
_COLS = 64
_CG = _COLS // 16
_STR = 80
_R = 128
_NCH = _N // _R
_NP = _NCH // 2


def _zero_acc(acc):
    z = jnp.zeros((16,), jnp.float32)

    def body(r, _):
        for g in range(_CG + 1):
            acc[pl.ds(r * _STR + g * 16, 16)] = z
        return 0
    lax.fori_loop(0, _NUM_CLASSES, body, 0)


def _feat_slice(feats, j, slice0):
    return feats.at[pl.ds(j * _R, _R), pl.ds(slice0, 128)]


def _accumulate_chunk(acc, fb, lab, base, coff):
    one = jnp.ones((16,), jnp.float32)

    def rowblk(rb, _):
        lv = lab[pl.ds(base + rb * 16, 16)]
        cbase = [lv[r] * _STR for r in range(16)]
        def load4(r4):
            return [[fb[rb * 16 + 4 * r4 + i, pl.ds(coff + g * 16, 16)]
                     for g in range(_CG)] for i in range(4)]

        def store4(r4, vals):
            for i in range(4):
                r = 4 * r4 + i
                for g in range(_CG):
                    plsc.addupdate(acc.at[pl.ds(cbase[r] + g * 16, 16)],
                                   vals[i][g])
                plsc.addupdate(acc.at[pl.ds(cbase[r] + _COLS, 16)], one)

        vals = load4(0)
        for r4 in range(1, 4):
            nxt = load4(r4)
            store4(r4 - 1, vals)
            vals = nxt
        store4(3, vals)
        return 0
    lax.fori_loop(0, _R // 16, rowblk, 0)


def _modality(feats, labels, out_hbm, lab, acc, fb0, fb1,
              sem0, sem1, sem2, w, slice0, coff):
    pltpu.async_copy(labels, lab, sem2)
    _zero_acc(acc)
    pltpu.make_async_copy(labels, lab, sem2).wait()
    pltpu.async_copy(_feat_slice(feats, 0, slice0), fb0, sem0)

    def pair(p, _):
        pltpu.async_copy(_feat_slice(feats, 2 * p + 1, slice0), fb1, sem1)
        pltpu.make_async_copy(_feat_slice(feats, 2 * p, slice0), fb0,
                              sem0).wait()
        _accumulate_chunk(acc, fb0, lab, 2 * p * _R, coff)

        @pl.when(p + 1 < _NP)
        def _():
            pltpu.async_copy(_feat_slice(feats, 2 * p + 2, slice0), fb0, sem0)
        pltpu.make_async_copy(_feat_slice(feats, 2 * p + 1, slice0), fb1,
                              sem1).wait()
        _accumulate_chunk(acc, fb1, lab, (2 * p + 1) * _R, coff)
        return 0
    lax.fori_loop(0, _NP, pair, 0)
    pltpu.sync_copy(acc, out_hbm.at[pl.ds(w * _NUM_CLASSES * _STR,
                                          _NUM_CLASSES * _STR)])


def _segsum_body(rgb_f, ir_f, rgb_l, ir_l, acc_v_hbm, acc_i_hbm,
                 lab, fb0, fb1, acc, sem0, sem1, sem2):
    c = lax.axis_index("c")
    s = lax.axis_index("s")
    w = s * 2 + c
    slice0 = (w // 2) * 128
    coff = (w % 2) * 64
    _modality(rgb_f, rgb_l, acc_v_hbm, lab, acc, fb0, fb1,
              sem0, sem1, sem2, w, slice0, coff)
    _modality(ir_f, ir_l, acc_i_hbm, lab, acc, fb0, fb1,
              sem0, sem1, sem2, w, slice0, coff)


def _segsum(rgb_feats, ir_feats, rgb_labels, ir_labels):
    mesh = plsc.VectorSubcoreMesh(core_axis_name="c", subcore_axis_name="s")
    run = pl.kernel(
        _segsum_body,
        out_type=(
            jax.ShapeDtypeStruct((_NW * _NUM_CLASSES * _STR,), jnp.float32),
            jax.ShapeDtypeStruct((_NW * _NUM_CLASSES * _STR,), jnp.float32),
        ),
        mesh=mesh,
        scratch_types=[
            pltpu.VMEM((_N,), jnp.int32),
            pltpu.VMEM((_R, 128), jnp.float32),
            pltpu.VMEM((_R, 128), jnp.float32),
            pltpu.VMEM((_NUM_CLASSES * _STR,), jnp.float32),
            pltpu.SemaphoreType.DMA,
            pltpu.SemaphoreType.DMA,
            pltpu.SemaphoreType.DMA,
        ],
    )
    return run(rgb_feats, ir_feats, rgb_labels, ir_labels)


_RB = 1000
_NRB = _NUM_CLASSES // _RB
_NWP = _NW // 2


def _ema_body(acc_v_ref, acc_i_ref, mem_v_ref, mem_i_ref, out_ref):
    for m, (acc_ref, mem_ref) in enumerate(
            ((acc_v_ref, mem_v_ref), (acc_i_ref, mem_i_ref))):
        acc = jnp.concatenate([acc_ref[0, :, :_COLS],
                               acc_ref[1, :, :_COLS]], axis=1)
        mem = mem_ref[...]
        cnt = acc_ref[0, :, _COLS:_COLS + 1]
        factor = _SIGMA / jnp.maximum(cnt, 1.0)
        upd = mem * (1.0 - _SIGMA) + acc * factor
        out_ref[m] = jnp.where(cnt > 0.0, upd, mem)


def _ema(acc_v, acc_i, vis_memory, ir_memory):
    acc_spec = pl.BlockSpec((2, _RB, _STR), lambda b, wp: (wp, b, 0))
    mem_spec = pl.BlockSpec((_RB, 128), lambda b, wp: (b, wp))
    return pl.pallas_call(
        _ema_body,
        grid=(_NRB, _NWP),
        in_specs=[
            acc_spec,
            acc_spec,
            mem_spec,
            mem_spec,
        ],
        out_specs=pl.BlockSpec((2, _RB, 128), lambda b, wp: (0, b, wp)),
        out_shape=jax.ShapeDtypeStruct((2, _NUM_CLASSES, _FEAT), jnp.float32),
    )(acc_v, acc_i, vis_memory, ir_memory)


@jax.jit
def _cma(rgb_feats, ir_feats, vis_memory, ir_memory, rgb_labels, ir_labels):
    acc_v, acc_i = _segsum(rgb_feats, ir_feats, rgb_labels, ir_labels)
    acc_v = acc_v.reshape(_NW, _NUM_CLASSES, _STR)
    acc_i = acc_i.reshape(_NW, _NUM_CLASSES, _STR)
    return _ema(acc_v, acc_i, vis_memory, ir_memory)


def kernel(rgb_feats, ir_feats, vis_memory, ir_memory, rgb_labels, ir_labels):
    return _cma(rgb_feats, ir_feats, vis_memory, ir_memory,
                rgb_labels.astype(jnp.int32), ir_labels.astype(jnp.int32))
